# baseline (speedup 1.0000x reference)
import functools

import jax
import jax.numpy as jnp
from jax import lax
from jax.experimental import pallas as pl
from jax.experimental.pallas import tpu as pltpu
from jax.experimental.pallas import tpu_sc as plsc

KP_EXTENT = 0.66
MIN_NORM = 1e-15
MAXNORM = 1.0 - 1e-5

D = 128
K = 4
NEI = 16

B1 = 1000
B3 = 200
GATHER_WIN = 128


def _artanh(z):
    z = jnp.clip(z, -1.0 + 1e-7, 1.0 - 1e-7)
    return 0.5 * jnp.log((1.0 + z) / (1.0 - z))


def _proj(z, n2=None):
    if n2 is None:
        n2 = jnp.sum(z * z, -1, keepdims=True)
    norm = jnp.maximum(jnp.sqrt(n2), MIN_NORM)
    scale = jnp.where(norm > MAXNORM, MAXNORM / norm, 1.0)
    return z * scale


def _pack2(a, b):
    au = lax.convert_element_type(
        lax.bitcast_convert_type(a.astype(jnp.bfloat16), jnp.uint16),
        jnp.uint32)
    bu = lax.convert_element_type(
        lax.bitcast_convert_type(b.astype(jnp.bfloat16), jnp.uint16),
        jnp.uint32)
    return lax.bitcast_convert_type(au | (bu << 16), jnp.float32)


def _unpack2(w):
    u = lax.bitcast_convert_type(w, jnp.uint32)
    a = lax.bitcast_convert_type(u << 16, jnp.float32)
    b = lax.bitcast_convert_type(u & jnp.uint32(0xFFFF0000), jnp.float32)
    return a, b


def _stage1_body(x_ref, w_ref, b_ref, kt_ref,
                 xm_ref, g01_ref, g23_ref, xk_ref, a2_ref):
    xb = x_ref[...]
    wf = w_ref[...]
    bf = b_ref[...]
    kt = kt_ref[...]

    x2 = jnp.sum(xb * xb, -1, keepdims=True)
    pn = jnp.maximum(jnp.sqrt(x2), MIN_NORM)
    t = (_artanh(pn) / pn) * xb

    bn = jnp.maximum(
        jnp.sqrt(jnp.sum(bf * bf, -1, keepdims=True)), MIN_NORM)
    hb = _proj((jnp.tanh(bn) / bn) * bf)
    hb2 = jnp.sum(hb * hb, -1, keepdims=True)

    packed = []
    gammas = []
    for k in range(K):
        u = jnp.dot(t, wf[k].T, preferred_element_type=jnp.float32)
        un2 = jnp.sum(u * u, -1, keepdims=True)
        un = jnp.maximum(jnp.sqrt(un2), MIN_NORM)
        feat = _proj((jnp.tanh(un) / un) * u)
        f2 = jnp.sum(feat * feat, -1, keepdims=True)
        fy = jnp.sum(feat * hb[k][None, :], -1, keepdims=True)
        y2 = hb2[k][None, :]
        num = (1.0 + 2.0 * fy + y2) * feat + (1.0 - f2) * hb[k][None, :]
        den = 1.0 + 2.0 * fy + f2 * y2
        m = _proj(num * (1.0 / jnp.maximum(den, MIN_NORM)))
        m2 = jnp.sum(m * m, -1, keepdims=True)
        kf = m * (2.0 / (1.0 + m2))
        k2 = jnp.sum(kf * kf, -1, keepdims=True)
        gam = lax.rsqrt(jnp.maximum(1.0 - k2, MIN_NORM))
        gammas.append(gam)
        gg = gam * kf
        packed.append(_pack2(gg[:, :64], gg[:, 64:]))
    g01_ref[...] = jnp.concatenate(packed[0:2], axis=-1)
    g23_ref[...] = jnp.concatenate(packed[2:4], axis=-1)

    xm_ref[:, 0:64] = _pack2(xb[:, :64], xb[:, 64:])
    xm_ref[:, 64:80] = jnp.concatenate(
        gammas + [x2] + [x2] * 11, axis=-1)
    xm_ref[:, 80:128] = jnp.zeros((B1, 48), jnp.float32)

    one_m_x2 = 1.0 - x2
    a2s = []
    for k in range(1, K):
        tmp = one_m_x2 * kt[k][None, :]
        t2 = jnp.sum(tmp * tmp, -1, keepdims=True)
        tmp = _proj(tmp, t2)
        t2 = jnp.sum(tmp * tmp, -1, keepdims=True)
        tn = jnp.maximum(jnp.sqrt(t2), MIN_NORM)
        lam = 2.0 / jnp.maximum(one_m_x2, MIN_NORM)
        second = (jnp.tanh(lam * tn / 2.0) / tn) * tmp
        s2 = jnp.sum(second * second, -1, keepdims=True)
        xs = jnp.sum(xb * second, -1, keepdims=True)
        num = (1.0 + 2.0 * xs + s2) * xb + (1.0 - x2) * second
        den = 1.0 + 2.0 * xs + x2 * s2
        xk = _proj(num * (1.0 / jnp.maximum(den, MIN_NORM)))
        xk_ref[:, (k - 1) * D:k * D] = xk
        a2s.append(jnp.sum(xk * xk, -1, keepdims=True))
    a2s.append(x2)
    a2_ref[...] = jnp.concatenate(a2s + a2s + a2s + a2s, axis=-1)


def _build_tables(x, W, b, kt):
    n = x.shape[0]
    plane = jax.ShapeDtypeStruct((n, D), jnp.float32)
    return pl.pallas_call(
        _stage1_body,
        grid=(n // B1,),
        in_specs=[
            pl.BlockSpec((B1, D), lambda i: (i, 0)),
            pl.BlockSpec((K, D, D), lambda i: (0, 0, 0)),
            pl.BlockSpec((K, D), lambda i: (0, 0)),
            pl.BlockSpec((K, D), lambda i: (0, 0)),
        ],
        out_specs=[
            pl.BlockSpec((B1, D), lambda i: (i, 0)),
            pl.BlockSpec((B1, D), lambda i: (i, 0)),
            pl.BlockSpec((B1, D), lambda i: (i, 0)),
            pl.BlockSpec((B1, 3 * D), lambda i: (i, 0)),
            pl.BlockSpec((B1, 16), lambda i: (i, 0)),
        ],
        out_shape=[plane, plane, plane,
                   jax.ShapeDtypeStruct((n, 3 * D), jnp.float32),
                   jax.ShapeDtypeStruct((n, 16), jnp.float32)],
        compiler_params=pltpu.CompilerParams(
            dimension_semantics=("parallel",)),
    )(x, W, b, kt)


def _sc_gather(xm, g01, g23, idx_flat):
    num_idx = idx_flat.shape[0]
    idx2 = idx_flat.reshape(1, num_idx)
    mesh = plsc.VectorSubcoreMesh(
        core_axis_name="core", subcore_axis_name="subcore")
    out = jax.ShapeDtypeStruct((num_idx, D), jnp.float32)

    @functools.partial(pl.kernel, out_type=[out, out, out], mesh=mesh)
    def gather_kernel(x_hbm, a_hbm, b_hbm, i_hbm, ox_hbm, oa_hbm, ob_hbm):
        def body(i_vmem, ox_v, oa_v, ob_v):
            pltpu.sync_copy(x_hbm.at[i_vmem.at[0]], ox_v)
            pltpu.sync_copy(a_hbm.at[i_vmem.at[0]], oa_v)
            pltpu.sync_copy(b_hbm.at[i_vmem.at[0]], ob_v)

        pltpu.emit_pipeline(
            body,
            grid=(num_idx // GATHER_WIN,),
            in_specs=[pl.BlockSpec((1, GATHER_WIN),
                                   index_map=lambda i: (0, i))],
            out_specs=[pl.BlockSpec((GATHER_WIN, D),
                                    index_map=lambda i: (i, 0))] * 3,
            core_axis_name=("core", "subcore"),
            dimension_semantics=(pltpu.PARALLEL,),
        )(i_hbm, ox_hbm, oa_hbm, ob_hbm)

    return gather_kernel(xm, g01, g23, idx2)


def _stage3_body(x_ref, mask_ref, xk_ref, a2_ref,
                 xm_ref, g01_ref, g23_ref, o_ref):
    xb = x_ref[...]
    mask = mask_ref[...]
    a2v = a2_ref[...]

    xm = xm_ref[...]
    lo, hi = _unpack2(xm[:, :64])
    xn = jnp.concatenate([lo, hi], axis=-1).reshape(B3, NEI, D)
    xm3 = xm.reshape(B3, NEI, D)
    y2 = xm3[:, :, 68]

    ays = []
    for k in range(K):
        a = (xk_ref[:, k * D:(k + 1) * D] if k < K - 1 else xb)
        ays.append(jnp.sum(a[:, None, :] * xn, -1))
    ay = jnp.concatenate(ays, axis=-1)
    a2 = jnp.concatenate(
        [jnp.broadcast_to(a2v[:, k:k + 1], (B3, NEI)) for k in range(K)],
        axis=-1)
    y2c = jnp.concatenate([y2] * K, axis=-1)
    A = 1.0 - 2.0 * ay + y2c
    Bc = 1.0 - a2
    nn2 = A * A * a2 + Bc * Bc * y2c - 2.0 * A * Bc * ay
    dd = jnp.maximum(1.0 - 2.0 * ay + a2 * y2c, MIN_NORM)
    un = jnp.sqrt(jnp.maximum(nn2, 0.0)) / dd
    u2 = un * un
    dis = 2.0 * un * (1.0 + u2 * (1.0 / 3.0 + u2 * (1.0 / 5.0 + u2 * (
        1.0 / 7.0 + u2 * (1.0 / 9.0)))))
    w = jnp.maximum(0.0, 1.0 - dis / KP_EXTENT) * \
        jnp.concatenate([mask] * K, axis=-1)

    gam = jnp.concatenate(
        [xm3[:, :, 64 + k] for k in range(K)], axis=-1)
    wg = (w * gam).reshape(B3, K, NEI)
    dens = jnp.sum(wg, axis=-1)

    g01 = g01_ref[...]
    g23 = g23_ref[...]
    num2 = jnp.zeros((B3, D), jnp.float32)
    den2 = jnp.zeros((B3, 1), jnp.float32)
    for k in range(K):
        src = g01 if k < 2 else g23
        lo, hi = _unpack2(src[:, (k % 2) * 64:(k % 2) * 64 + 64])
        gg = jnp.concatenate([lo, hi], axis=-1).reshape(B3, NEI, D)
        wk = w[:, k * NEI:(k + 1) * NEI]
        num_k = jnp.sum(wk[:, :, None] * gg, axis=1)
        den_k = jnp.maximum(dens[:, k:k + 1], MIN_NORM)
        mid = num_k * (1.0 / den_k)
        m2 = jnp.sum(mid * mid, -1, keepdims=True)
        g2 = lax.rsqrt(jnp.maximum(1.0 - m2, MIN_NORM))
        num2 = num2 + g2 * mid
        den2 = den2 + g2
    midk = num2 * (1.0 / jnp.maximum(den2, MIN_NORM))
    mk2 = jnp.sum(midk * midk, -1, keepdims=True)
    p = midk * (1.0 / (1.0 + jnp.sqrt(jnp.maximum(1.0 - mk2, MIN_NORM))))
    o_ref[...] = _proj(p)


def _aggregate(x, nei_mask, xk, a2s, xm, g01, g23):
    n = x.shape[0]
    gspec = pl.BlockSpec((B3 * NEI, D), lambda i: (i, 0))
    return pl.pallas_call(
        _stage3_body,
        grid=(n // B3,),
        in_specs=[
            pl.BlockSpec((B3, D), lambda i: (i, 0)),
            pl.BlockSpec((B3, NEI), lambda i: (i, 0)),
            pl.BlockSpec((B3, 3 * D), lambda i: (i, 0)),
            pl.BlockSpec((B3, 16), lambda i: (i, 0)),
            gspec, gspec, gspec,
        ],
        out_specs=pl.BlockSpec((B3, D), lambda i: (i, 0)),
        out_shape=jax.ShapeDtypeStruct((n, D), jnp.float32),
        compiler_params=pltpu.CompilerParams(
            dimension_semantics=("parallel",)),
    )(x, nei_mask, xk, a2s, xm, g01, g23)


CHUNKS = 10


def kernel(x, nei, nei_mask, kernel_tangents, W, b):
    n = x.shape[0]
    xm, g01, g23, xk, a2s = _build_tables(x, W, b, kernel_tangents)
    nei_flat = nei.reshape(n * NEI)
    c = n // CHUNKS
    outs = []
    for i in range(CHUNKS):
        gx, ga, gb = _sc_gather(
            xm, g01, g23, nei_flat[i * c * NEI:(i + 1) * c * NEI])
        outs.append(_aggregate(
            x[i * c:(i + 1) * c], nei_mask[i * c:(i + 1) * c],
            xk[i * c:(i + 1) * c], a2s[i * c:(i + 1) * c], gx, ga, gb))
    return jnp.concatenate(outs, axis=0)


# speedup vs baseline: 1.0566x; 1.0566x over previous
import functools

import jax
import jax.numpy as jnp
from jax import lax
from jax.experimental import pallas as pl
from jax.experimental.pallas import tpu as pltpu
from jax.experimental.pallas import tpu_sc as plsc

KP_EXTENT = 0.66
MIN_NORM = 1e-15
MAXNORM = 1.0 - 1e-5

D = 128
K = 4
NEI = 16

B1 = 1000
B3 = 400
GATHER_WIN = 128


def _artanh(z):
    z = jnp.clip(z, -1.0 + 1e-7, 1.0 - 1e-7)
    return 0.5 * jnp.log((1.0 + z) / (1.0 - z))


def _proj(z, n2=None):
    if n2 is None:
        n2 = jnp.sum(z * z, -1, keepdims=True)
    norm = jnp.maximum(jnp.sqrt(n2), MIN_NORM)
    scale = jnp.where(norm > MAXNORM, MAXNORM / norm, 1.0)
    return z * scale


def _pack2(a, b):
    au = lax.convert_element_type(
        lax.bitcast_convert_type(a.astype(jnp.bfloat16), jnp.uint16),
        jnp.uint32)
    bu = lax.convert_element_type(
        lax.bitcast_convert_type(b.astype(jnp.bfloat16), jnp.uint16),
        jnp.uint32)
    return lax.bitcast_convert_type(au | (bu << 16), jnp.float32)


def _unpack2(w):
    u = lax.bitcast_convert_type(w, jnp.uint32)
    a = lax.bitcast_convert_type(u << 16, jnp.float32)
    b = lax.bitcast_convert_type(u & jnp.uint32(0xFFFF0000), jnp.float32)
    return a, b


def _stage1_body(x_ref, w_ref, b_ref, kt_ref,
                 xm_ref, g01_ref, g23_ref, xk_ref, a2_ref):
    xb = x_ref[...]
    wf = w_ref[...]
    bf = b_ref[...]
    kt = kt_ref[...]

    x2 = jnp.sum(xb * xb, -1, keepdims=True)
    pn = jnp.maximum(jnp.sqrt(x2), MIN_NORM)
    t = (_artanh(pn) / pn) * xb

    bn = jnp.maximum(
        jnp.sqrt(jnp.sum(bf * bf, -1, keepdims=True)), MIN_NORM)
    hb = _proj((jnp.tanh(bn) / bn) * bf)
    hb2 = jnp.sum(hb * hb, -1, keepdims=True)

    packed = []
    gammas = []
    for k in range(K):
        u = jnp.dot(t, wf[k].T, preferred_element_type=jnp.float32)
        un2 = jnp.sum(u * u, -1, keepdims=True)
        un = jnp.maximum(jnp.sqrt(un2), MIN_NORM)
        feat = _proj((jnp.tanh(un) / un) * u)
        f2 = jnp.sum(feat * feat, -1, keepdims=True)
        fy = jnp.sum(feat * hb[k][None, :], -1, keepdims=True)
        y2 = hb2[k][None, :]
        num = (1.0 + 2.0 * fy + y2) * feat + (1.0 - f2) * hb[k][None, :]
        den = 1.0 + 2.0 * fy + f2 * y2
        m = _proj(num * (1.0 / jnp.maximum(den, MIN_NORM)))
        m2 = jnp.sum(m * m, -1, keepdims=True)
        kf = m * (2.0 / (1.0 + m2))
        k2 = jnp.sum(kf * kf, -1, keepdims=True)
        gam = lax.rsqrt(jnp.maximum(1.0 - k2, MIN_NORM))
        gammas.append(gam)
        gg = gam * kf
        packed.append(_pack2(gg[:, :64], gg[:, 64:]))
    g01_ref[...] = jnp.concatenate(packed[0:2], axis=-1)
    g23_ref[...] = jnp.concatenate(packed[2:4], axis=-1)

    xm_ref[:, 0:64] = _pack2(xb[:, :64], xb[:, 64:])
    xm_ref[:, 64:80] = jnp.concatenate(
        gammas + [x2] + [x2] * 11, axis=-1)
    xm_ref[:, 80:128] = jnp.zeros((B1, 48), jnp.float32)

    one_m_x2 = 1.0 - x2
    a2s = []
    for k in range(1, K):
        tmp = one_m_x2 * kt[k][None, :]
        t2 = jnp.sum(tmp * tmp, -1, keepdims=True)
        tmp = _proj(tmp, t2)
        t2 = jnp.sum(tmp * tmp, -1, keepdims=True)
        tn = jnp.maximum(jnp.sqrt(t2), MIN_NORM)
        lam = 2.0 / jnp.maximum(one_m_x2, MIN_NORM)
        second = (jnp.tanh(lam * tn / 2.0) / tn) * tmp
        s2 = jnp.sum(second * second, -1, keepdims=True)
        xs = jnp.sum(xb * second, -1, keepdims=True)
        num = (1.0 + 2.0 * xs + s2) * xb + (1.0 - x2) * second
        den = 1.0 + 2.0 * xs + x2 * s2
        xk = _proj(num * (1.0 / jnp.maximum(den, MIN_NORM)))
        xk_ref[:, (k - 1) * D:k * D] = xk
        a2s.append(jnp.sum(xk * xk, -1, keepdims=True))
    a2s.append(x2)
    a2_ref[...] = jnp.concatenate(a2s + a2s + a2s + a2s, axis=-1)


def _build_tables(x, W, b, kt):
    n = x.shape[0]
    plane = jax.ShapeDtypeStruct((n, D), jnp.float32)
    return pl.pallas_call(
        _stage1_body,
        grid=(n // B1,),
        in_specs=[
            pl.BlockSpec((B1, D), lambda i: (i, 0)),
            pl.BlockSpec((K, D, D), lambda i: (0, 0, 0)),
            pl.BlockSpec((K, D), lambda i: (0, 0)),
            pl.BlockSpec((K, D), lambda i: (0, 0)),
        ],
        out_specs=[
            pl.BlockSpec((B1, D), lambda i: (i, 0)),
            pl.BlockSpec((B1, D), lambda i: (i, 0)),
            pl.BlockSpec((B1, D), lambda i: (i, 0)),
            pl.BlockSpec((B1, 3 * D), lambda i: (i, 0)),
            pl.BlockSpec((B1, 16), lambda i: (i, 0)),
        ],
        out_shape=[plane, plane, plane,
                   jax.ShapeDtypeStruct((n, 3 * D), jnp.float32),
                   jax.ShapeDtypeStruct((n, 16), jnp.float32)],
        compiler_params=pltpu.CompilerParams(
            dimension_semantics=("parallel",)),
    )(x, W, b, kt)


def _sc_gather(xm, g01, g23, idx_flat):
    num_idx = idx_flat.shape[0]
    idx2 = idx_flat.reshape(1, num_idx)
    mesh = plsc.VectorSubcoreMesh(
        core_axis_name="core", subcore_axis_name="subcore")
    out = jax.ShapeDtypeStruct((num_idx, D), jnp.float32)

    @functools.partial(pl.kernel, out_type=[out, out, out], mesh=mesh)
    def gather_kernel(x_hbm, a_hbm, b_hbm, i_hbm, ox_hbm, oa_hbm, ob_hbm):
        def body(i_vmem, ox_v, oa_v, ob_v):
            pltpu.sync_copy(x_hbm.at[i_vmem.at[0]], ox_v)
            pltpu.sync_copy(a_hbm.at[i_vmem.at[0]], oa_v)
            pltpu.sync_copy(b_hbm.at[i_vmem.at[0]], ob_v)

        pltpu.emit_pipeline(
            body,
            grid=(num_idx // GATHER_WIN,),
            in_specs=[pl.BlockSpec((1, GATHER_WIN),
                                   index_map=lambda i: (0, i))],
            out_specs=[pl.BlockSpec((GATHER_WIN, D),
                                    index_map=lambda i: (i, 0))] * 3,
            core_axis_name=("core", "subcore"),
            dimension_semantics=(pltpu.PARALLEL,),
        )(i_hbm, ox_hbm, oa_hbm, ob_hbm)

    return gather_kernel(xm, g01, g23, idx2)


def _stage3_body(x_ref, mask_ref, xk_ref, a2_ref,
                 xm_ref, g01_ref, g23_ref, o_ref):
    xb = x_ref[...]
    mask = mask_ref[...]
    a2v = a2_ref[...]

    xm = xm_ref[...]
    lo, hi = _unpack2(xm[:, :64])
    xn = jnp.concatenate([lo, hi], axis=-1).reshape(B3, NEI, D)
    xm3 = xm.reshape(B3, NEI, D)
    y2 = xm3[:, :, 68]

    ays = []
    for k in range(K):
        a = (xk_ref[:, k * D:(k + 1) * D] if k < K - 1 else xb)
        ays.append(jnp.sum(a[:, None, :] * xn, -1))
    ay = jnp.concatenate(ays, axis=-1)
    a2 = jnp.concatenate(
        [jnp.broadcast_to(a2v[:, k:k + 1], (B3, NEI)) for k in range(K)],
        axis=-1)
    y2c = jnp.concatenate([y2] * K, axis=-1)
    A = 1.0 - 2.0 * ay + y2c
    Bc = 1.0 - a2
    nn2 = A * A * a2 + Bc * Bc * y2c - 2.0 * A * Bc * ay
    dd = jnp.maximum(1.0 - 2.0 * ay + a2 * y2c, MIN_NORM)
    un = jnp.sqrt(jnp.maximum(nn2, 0.0)) / dd
    u2 = un * un
    dis = 2.0 * un * (1.0 + u2 * (1.0 / 3.0 + u2 * (1.0 / 5.0 + u2 * (
        1.0 / 7.0 + u2 * (1.0 / 9.0)))))
    w = jnp.maximum(0.0, 1.0 - dis / KP_EXTENT) * \
        jnp.concatenate([mask] * K, axis=-1)

    gam = jnp.concatenate(
        [xm3[:, :, 64 + k] for k in range(K)], axis=-1)
    wg = (w * gam).reshape(B3, K, NEI)
    dens = jnp.sum(wg, axis=-1)

    g01 = g01_ref[...]
    g23 = g23_ref[...]
    num2 = jnp.zeros((B3, D), jnp.float32)
    den2 = jnp.zeros((B3, 1), jnp.float32)
    for k in range(K):
        src = g01 if k < 2 else g23
        lo, hi = _unpack2(src[:, (k % 2) * 64:(k % 2) * 64 + 64])
        gg = jnp.concatenate([lo, hi], axis=-1).reshape(B3, NEI, D)
        wk = w[:, k * NEI:(k + 1) * NEI]
        num_k = jnp.sum(wk[:, :, None] * gg, axis=1)
        den_k = jnp.maximum(dens[:, k:k + 1], MIN_NORM)
        mid = num_k * (1.0 / den_k)
        m2 = jnp.sum(mid * mid, -1, keepdims=True)
        g2 = lax.rsqrt(jnp.maximum(1.0 - m2, MIN_NORM))
        num2 = num2 + g2 * mid
        den2 = den2 + g2
    midk = num2 * (1.0 / jnp.maximum(den2, MIN_NORM))
    mk2 = jnp.sum(midk * midk, -1, keepdims=True)
    p = midk * (1.0 / (1.0 + jnp.sqrt(jnp.maximum(1.0 - mk2, MIN_NORM))))
    o_ref[...] = _proj(p)


def _aggregate(x, nei_mask, xk, a2s, xm, g01, g23):
    n = x.shape[0]
    gspec = pl.BlockSpec((B3 * NEI, D), lambda i: (i, 0))
    return pl.pallas_call(
        _stage3_body,
        grid=(n // B3,),
        in_specs=[
            pl.BlockSpec((B3, D), lambda i: (i, 0)),
            pl.BlockSpec((B3, NEI), lambda i: (i, 0)),
            pl.BlockSpec((B3, 3 * D), lambda i: (i, 0)),
            pl.BlockSpec((B3, 16), lambda i: (i, 0)),
            gspec, gspec, gspec,
        ],
        out_specs=pl.BlockSpec((B3, D), lambda i: (i, 0)),
        out_shape=jax.ShapeDtypeStruct((n, D), jnp.float32),
        compiler_params=pltpu.CompilerParams(
            dimension_semantics=("parallel",)),
    )(x, nei_mask, xk, a2s, xm, g01, g23)


CHUNKS = 5


def kernel(x, nei, nei_mask, kernel_tangents, W, b):
    n = x.shape[0]
    xm, g01, g23, xk, a2s = _build_tables(x, W, b, kernel_tangents)
    nei_flat = nei.reshape(n * NEI)
    c = n // CHUNKS
    outs = []
    for i in range(CHUNKS):
        gx, ga, gb = _sc_gather(
            xm, g01, g23, nei_flat[i * c * NEI:(i + 1) * c * NEI])
        outs.append(_aggregate(
            x[i * c:(i + 1) * c], nei_mask[i * c:(i + 1) * c],
            xk[i * c:(i + 1) * c], a2s[i * c:(i + 1) * c], gx, ga, gb))
    return jnp.concatenate(outs, axis=0)
